# Initial kernel scaffold; baseline (speedup 1.0000x reference)
import jax
import jax.numpy as jnp
from jax.experimental import pallas as pl


def kernel(x, edge_index, pos, W1, b1, W2, b2, D1, dB1, D2, dB2, FW, Fb, gamma, beta):
    raise NotImplementedError("write your pallas kernel here")


# speedup vs baseline: 8.7494x; 8.7494x over previous
import functools

import jax
import jax.numpy as jnp
from jax import lax
from jax.experimental import pallas as pl
from jax.experimental.pallas import tpu as pltpu
from jax.experimental.pallas import tpu_sc as plsc

N = 10000
E = 320000
IN = 128
OUT = 128
S = 4
PS = OUT // S
TW = 144
AW = 144
SCALES = (5.0, 10.0, 25.0, 50.0)

NC = 2
NS = 16
NW = NC * NS
CH = 64
NCHUNK = E // CH
BASE_CHUNKS = NCHUNK // NW
EXTRA = NCHUNK - BASE_CHUNKS * NW
AGGN = 10240
ROWS_PER_TILE = AGGN // NS

_MAGIC = 0x5F3759DF


def _pre_body(x_ref, pos16_ref, w1ab_ref, ta_ref, tb_ref):
    p = jnp.dot(x_ref[...], w1ab_ref[...], preferred_element_type=jnp.float32)
    pos = pos16_ref[...]
    ta_ref[...] = jnp.concatenate([p[:, :IN], pos], axis=1)
    tb_ref[...] = jnp.concatenate([p[:, IN:], pos], axis=1)


def _pre_call(x, pos16, w1ab):
    B = 2000
    return pl.pallas_call(
        _pre_body,
        grid=(N // B,),
        in_specs=[
            pl.BlockSpec((B, IN), lambda i: (i, 0)),
            pl.BlockSpec((B, 16), lambda i: (i, 0)),
            pl.BlockSpec((IN, 2 * IN), lambda i: (0, 0)),
        ],
        out_specs=[
            pl.BlockSpec((B, TW), lambda i: (i, 0)),
            pl.BlockSpec((B, TW), lambda i: (i, 0)),
        ],
        out_shape=[jax.ShapeDtypeStruct((N, TW), jnp.float32)] * 2,
    )(x, pos16, w1ab)


def _sc_body(ta, tb, ei, smallw_h, evw_h, out, agg, idxv, bufA, bufB,
             wv, evw):
    c = lax.axis_index("c")
    s = lax.axis_index("s")
    wid = s * NC + c

    pltpu.sync_copy(smallw_h, wv)
    pltpu.sync_copy(evw_h, evw)

    zero16 = jnp.zeros((16,), jnp.float32)

    def zrow(i, carry):
        for j in range(AW // 16):
            bufA[i, pl.ds(j * 16, 16)] = zero16
        return carry

    lax.fori_loop(0, CH, zrow, 0)

    base_row = s * ROWS_PER_TILE
    for i in range(ROWS_PER_TILE // CH):
        pltpu.sync_copy(bufA, agg.at[pl.ds(base_row + i * CH, CH)])
    plsc.subcore_barrier()

    c1v = [wv[q, 0, :] for q in range(S)]
    db1v = [wv[q, 1, :] for q in range(S)]
    d2v = [wv[q, 2, :] for q in range(S)]
    db2s = [wv[q, 3, :][0] for q in range(S)]
    w1lv = [evw[0, pl.ds(i * 16, 16)] for i in range(8)]
    b1v = [evw[1, pl.ds(i * 16, 16)] for i in range(8)]
    lanes = lax.iota(jnp.int32, 16)
    onehot = jnp.where(lanes == 0, 1.0, 0.0).astype(jnp.float32)

    nch = BASE_CHUNKS + jnp.where(wid < EXTRA, 1, 0)

    def chunk_body(j, carry):
        cid = wid + j * NW
        base = cid * CH
        pltpu.sync_copy(ei.at[1, pl.ds(base, CH)], idxv.at[0])
        pltpu.sync_copy(ei.at[0, pl.ds(base, CH)], idxv.at[1])
        pltpu.sync_copy(ta.at[idxv.at[0]], bufA)
        pltpu.sync_copy(tb.at[idxv.at[1]], bufB)

        def group_body(g, gcarry):
            ridx = lanes + g * 16

            def gat(buf, colx):
                cvec = jnp.full((16,), colx, jnp.int32)
                return plsc.load_gather(buf, [ridx, cvec])

            dx = gat(bufA, IN) - gat(bufB, IN)
            dy = gat(bufA, IN + 1) - gat(bufB, IN + 1)
            dz = gat(bufA, IN + 2) - gat(bufB, IN + 2)
            d2 = dx * dx + dy * dy + dz * dz
            d2c = jnp.maximum(d2, 1e-30)
            ibits = _MAGIC - lax.shift_right_arithmetic(
                plsc.bitcast(d2c, jnp.int32), 1)
            y = plsc.bitcast(ibits, jnp.float32)
            for _ in range(3):
                y = y * (1.5 - 0.5 * d2c * y * y)
            distv = d2 * y

            for l in range(16):
                e = g * 16 + l
                dist = distv[l]
                for q in range(S):
                    t = jnp.maximum(dist * c1v[q] + db1v[q], 0.0)
                    dfs = jnp.sum(t * d2v[q]) + db2s[q]
                    for half in range(2):
                        i = q * 2 + half
                        va = bufA[e, pl.ds(i * 16, 16)]
                        vb = bufB[e, pl.ds(i * 16, 16)]
                        v = jnp.maximum(va + vb + dfs * w1lv[i] + b1v[i], 0.0)
                        bufA[e, pl.ds(i * 16, 16)] = v
                bufA[e, pl.ds(IN, 16)] = onehot
            return gcarry

        lax.fori_loop(0, CH // 16, group_body, 0)

        pltpu.sync_copy(bufA, agg.at[idxv.at[0]], add=True)
        return carry

    lax.fori_loop(0, nch, chunk_body, 0)
    plsc.subcore_barrier()

    for i in range(ROWS_PER_TILE // CH):
        pltpu.sync_copy(agg.at[pl.ds(base_row + i * CH, CH)],
                        out.at[c, pl.ds(base_row + i * CH, CH)])


def _sc_call(ta, tb, ei, smallw, evw):
    mesh = plsc.VectorSubcoreMesh(core_axis_name="c", subcore_axis_name="s")
    run = pl.kernel(
        _sc_body,
        out_type=jax.ShapeDtypeStruct((NC, AGGN, AW), jnp.float32),
        mesh=mesh,
        compiler_params=pltpu.CompilerParams(use_tc_tiling_on_sc=False,
                                             needs_layout_passes=False),
        scratch_types=[
            pltpu.VMEM_SHARED((AGGN, AW), jnp.float32),
            pltpu.VMEM((2, CH), jnp.int32),
            pltpu.VMEM((CH, TW), jnp.float32),
            pltpu.VMEM((CH, TW), jnp.float32),
            pltpu.VMEM((S, 4, 16), jnp.float32),
            pltpu.VMEM((2, IN), jnp.float32),
        ],
    )
    return run(ta, tb, ei, smallw, evw)


def _post_body(g0, g1, w2bd, b2c, fw, fb, gamma, beta, o_ref):
    a = g0[...] + g1[...]
    gs = a[:, :OUT]
    cnt = a[:, OUT:OUT + 1]
    m = gs / jnp.maximum(cnt, 1.0)
    ms = (jnp.dot(m, w2bd[...], preferred_element_type=jnp.float32)
          + jnp.where(cnt > 0.0, 1.0, 0.0) * b2c[...])
    y = jnp.maximum(
        jnp.dot(ms, fw[...], preferred_element_type=jnp.float32) + fb[...], 0.0)
    mu = jnp.mean(y, axis=1, keepdims=True)
    var = jnp.mean(jnp.square(y - mu), axis=1, keepdims=True)
    o_ref[...] = (y - mu) * lax.rsqrt(var + 1e-5) * gamma[...] + beta[...]


def _post_call(g0, g1, w2bd, b2c, fw, fb, gamma, beta):
    B = 2000
    vec = lambda: pl.BlockSpec((1, OUT), lambda i: (0, 0))
    return pl.pallas_call(
        _post_body,
        grid=(N // B,),
        in_specs=[
            pl.BlockSpec((B, AW), lambda i: (i, 0)),
            pl.BlockSpec((B, AW), lambda i: (i, 0)),
            pl.BlockSpec((OUT, OUT), lambda i: (0, 0)),
            vec(),
            pl.BlockSpec((OUT, OUT), lambda i: (0, 0)),
            vec(), vec(), vec(),
        ],
        out_specs=pl.BlockSpec((B, OUT), lambda i: (i, 0)),
        out_shape=jax.ShapeDtypeStruct((N, OUT), jnp.float32),
    )(g0, g1, w2bd, b2c, fw, fb, gamma, beta)


def kernel(x, edge_index, pos, W1, b1, W2, b2, D1, dB1, D2, dB2, FW, Fb,
           gamma, beta):
    scales = jnp.array(SCALES, jnp.float32)
    w1a = jnp.concatenate([W1[q, :IN, :] for q in range(S)], axis=1)
    w1b = jnp.concatenate([W1[q, IN:2 * IN, :] for q in range(S)], axis=1)
    w1ab = jnp.concatenate([w1a, w1b], axis=1)
    c1 = D1[:, 0, :] / scales[:, None]
    d2w = D2[:, :, 0]
    db2p = jnp.pad(dB2, ((0, 0), (0, 15)))
    smallw = jnp.stack([c1, dB1, d2w, db2p], axis=1)
    evw = jnp.stack([W1[:, 2 * IN, :].reshape(-1), b1.reshape(-1)])
    pos16 = jnp.concatenate([pos, jnp.zeros((N, 13), jnp.float32)], axis=1)

    ta, tb = _pre_call(x, pos16, w1ab)
    aggout = _sc_call(ta, tb, edge_index, smallw, evw)

    w2bd = jnp.zeros((OUT, OUT), jnp.float32)
    for q in range(S):
        w2bd = w2bd.at[q * PS:(q + 1) * PS, q * PS:(q + 1) * PS].set(W2[q])
    b2c = b2.reshape(1, -1)
    return _post_call(aggout[0, :N], aggout[1, :N], w2bd, b2c, FW,
                      Fb.reshape(1, -1), gamma.reshape(1, -1),
                      beta.reshape(1, -1))
